# baseline (speedup 1.0000x reference)
import jax
import jax.numpy as jnp
from jax import lax
from jax.experimental import pallas as pl
from jax.experimental.pallas import tpu as pltpu

_BN = 256


def _dot_nt(a, b):
    return lax.dot_general(
        a, b,
        dimension_numbers=(((1,), (1,)), ((), ())),
        preferred_element_type=jnp.float32,
        precision=lax.Precision.DEFAULT,
    )


def _linear_kernel(xa_ref, xb_ref, wa_ref, wb_ref, b_ref, o_ref):
    n = pl.program_id(0)
    half = xa_ref.shape[0]

    def compute(w_ref):
        o_ref[:half, :] = _dot_nt(xa_ref[...], w_ref[...]) + b_ref[...]
        o_ref[half:, :] = _dot_nt(xb_ref[...], w_ref[...]) + b_ref[...]

    @pl.when(n % 2 == 0)
    def _():
        compute(wa_ref)

    @pl.when(n % 2 == 1)
    def _():
        compute(wb_ref)


def kernel(x, weight, bias):
    batch, in_f = x.shape
    out_f = weight.shape[0]
    half = batch // 2
    brow = bias.reshape(1, out_f)
    return pl.pallas_call(
        _linear_kernel,
        grid=(out_f // _BN,),
        in_specs=[
            pl.BlockSpec((half, in_f), lambda n: (0, 0),
                         pipeline_mode=pl.Buffered(buffer_count=1)),
            pl.BlockSpec((half, in_f), lambda n: (1, 0),
                         pipeline_mode=pl.Buffered(buffer_count=1)),
            pl.BlockSpec((_BN, in_f), lambda n: (2 * (n // 2), 0)),
            pl.BlockSpec((_BN, in_f), lambda n: (2 * (n // 2) + 1, 0)),
            pl.BlockSpec((1, _BN), lambda n: (0, n)),
        ],
        out_specs=pl.BlockSpec((batch, _BN), lambda n: (0, n)),
        out_shape=jax.ShapeDtypeStruct((batch, out_f), jnp.float32),
        compiler_params=pltpu.CompilerParams(
            dimension_semantics=("arbitrary",),
        ),
    )(x, x, weight, weight, brow)


# speedup vs baseline: 1.2355x; 1.2355x over previous
import jax
import jax.numpy as jnp
from jax import lax
from jax.experimental import pallas as pl
from jax.experimental.pallas import tpu as pltpu

_BN = 512


def _dot_nt(a, b):
    return lax.dot_general(
        a, b,
        dimension_numbers=(((1,), (1,)), ((), ())),
        preferred_element_type=jnp.float32,
        precision=lax.Precision.DEFAULT,
    )


def _linear_kernel(x_ref, wl_ref, wr_ref, b_ref, o_ref):
    kh = wl_ref.shape[1]
    acc = _dot_nt(x_ref[:, :kh], wl_ref[...])
    acc += _dot_nt(x_ref[:, kh:], wr_ref[...])
    o_ref[...] = acc + b_ref[...]


def kernel(x, weight, bias):
    batch, in_f = x.shape
    out_f = weight.shape[0]
    kh = in_f // 2
    brow = bias.reshape(1, out_f)
    return pl.pallas_call(
        _linear_kernel,
        grid=(out_f // _BN,),
        in_specs=[
            pl.BlockSpec((batch, in_f), lambda n: (0, 0),
                         pipeline_mode=pl.Buffered(buffer_count=1)),
            pl.BlockSpec((_BN, kh), lambda n: (n, 0)),
            pl.BlockSpec((_BN, kh), lambda n: (n, 1)),
            pl.BlockSpec((1, _BN), lambda n: (0, n)),
        ],
        out_specs=pl.BlockSpec((batch, _BN), lambda n: (0, n)),
        out_shape=jax.ShapeDtypeStruct((batch, out_f), jnp.float32),
        compiler_params=pltpu.CompilerParams(
            dimension_semantics=("arbitrary",),
        ),
    )(x, weight, weight, brow)


# speedup vs baseline: 1.3025x; 1.0543x over previous
import jax
import jax.numpy as jnp
from jax import lax
from jax.experimental import pallas as pl
from jax.experimental.pallas import tpu as pltpu

_BN = 512
_MB = 4


def _dot_nt(a, b):
    return lax.dot_general(
        a, b,
        dimension_numbers=(((1,), (1,)), ((), ())),
        preferred_element_type=jnp.float32,
        precision=lax.Precision.DEFAULT,
    )


def _linear_kernel(x_hbm, w_ref, b_ref, o_ref, xs_ref, sem_x):
    n = pl.program_id(0)
    batch = xs_ref.shape[0]
    bm = batch // _MB

    def x_copy(m):
        return pltpu.make_async_copy(
            x_hbm.at[pl.ds(m * bm, bm), :],
            xs_ref.at[pl.ds(m * bm, bm), :],
            sem_x.at[m])

    @pl.when(n == 0)
    def _():
        for m in range(_MB):
            x_copy(m).start()
        for m in range(_MB):
            x_copy(m).wait()
            o_ref[pl.ds(m * bm, bm), :] = (
                _dot_nt(xs_ref[pl.ds(m * bm, bm), :], w_ref[...]) + b_ref[...])

    @pl.when(n > 0)
    def _():
        o_ref[...] = _dot_nt(xs_ref[...], w_ref[...]) + b_ref[...]


def kernel(x, weight, bias):
    batch, in_f = x.shape
    out_f = weight.shape[0]
    brow = bias.reshape(1, out_f)
    return pl.pallas_call(
        _linear_kernel,
        grid=(out_f // _BN,),
        in_specs=[
            pl.BlockSpec(memory_space=pl.ANY),
            pl.BlockSpec((_BN, in_f), lambda n: (n, 0)),
            pl.BlockSpec((1, _BN), lambda n: (0, n)),
        ],
        out_specs=pl.BlockSpec((batch, _BN), lambda n: (0, n)),
        out_shape=jax.ShapeDtypeStruct((batch, out_f), jnp.float32),
        scratch_shapes=[
            pltpu.VMEM((batch, in_f), jnp.float32),
            pltpu.SemaphoreType.DMA((_MB,)),
        ],
        compiler_params=pltpu.CompilerParams(
            dimension_semantics=("arbitrary",),
        ),
    )(x, weight, brow)
